# baseline (speedup 1.0000x reference)
import jax
import jax.numpy as jnp
from jax.experimental import pallas as pl
from jax.experimental.pallas import tpu as pltpu

B = 16384
N = 1000
BLOCK_B = 256


def _fused_body(x_ref, g_ref, out_ref):
    z = x_ref[...] + g_ref[...]
    m = jnp.max(z, axis=1, keepdims=True)
    ssum = jnp.sum(jnp.exp(z - m), axis=1, keepdims=True)
    s = 1.0 / ssum
    val = (1.0 - s) + s

    cols = jax.lax.broadcasted_iota(jnp.int32, z.shape, 1)
    idx = jnp.min(jnp.where(z == m, cols, N), axis=1, keepdims=True)
    out = jnp.where(cols == idx, val, 0.0)

    rows = jax.lax.broadcasted_iota(jnp.int32, z.shape, 0) + pl.program_id(0) * BLOCK_B
    out_ref[...] = jnp.where((rows == 0) & (cols == 1), 1.0, out)


def kernel(x, gumbels):
    return pl.pallas_call(
        _fused_body,
        grid=(B // BLOCK_B,),
        in_specs=[
            pl.BlockSpec((BLOCK_B, N), lambda i: (i, 0)),
            pl.BlockSpec((BLOCK_B, N), lambda i: (i, 0)),
        ],
        out_specs=pl.BlockSpec((BLOCK_B, N), lambda i: (i, 0)),
        out_shape=jax.ShapeDtypeStruct((B, N), jnp.float32),
        compiler_params=pltpu.CompilerParams(
            dimension_semantics=("parallel",),
        ),
    )(x, gumbels)


# speedup vs baseline: 1.0871x; 1.0871x over previous
import jax
import jax.numpy as jnp
from jax.experimental import pallas as pl
from jax.experimental.pallas import tpu as pltpu

B = 16384
N = 1000
BLOCK_B = 1024


def _fused_body(x_ref, g_ref, out_ref):
    z = x_ref[...] + g_ref[...]
    m = jnp.max(z, axis=1, keepdims=True)
    ssum = jnp.sum(jnp.exp(z - m), axis=1, keepdims=True)
    s = 1.0 / ssum
    val = (1.0 - s) + s

    cols = jax.lax.broadcasted_iota(jnp.int32, z.shape, 1)
    idx = jnp.min(jnp.where(z == m, cols, N), axis=1, keepdims=True)
    out = jnp.where(cols == idx, val, 0.0)

    rows = jax.lax.broadcasted_iota(jnp.int32, z.shape, 0) + pl.program_id(0) * BLOCK_B
    out_ref[...] = jnp.where((rows == 0) & (cols == 1), 1.0, out)


def kernel(x, gumbels):
    return pl.pallas_call(
        _fused_body,
        grid=(B // BLOCK_B,),
        in_specs=[
            pl.BlockSpec((BLOCK_B, N), lambda i: (i, 0)),
            pl.BlockSpec((BLOCK_B, N), lambda i: (i, 0)),
        ],
        out_specs=pl.BlockSpec((BLOCK_B, N), lambda i: (i, 0)),
        out_shape=jax.ShapeDtypeStruct((B, N), jnp.float32),
        compiler_params=pltpu.CompilerParams(
            dimension_semantics=("parallel",),
        ),
    )(x, gumbels)
